# baseline (speedup 1.0000x reference)
import functools

import jax
import jax.numpy as jnp
from jax import lax
from jax.experimental import pallas as pl
from jax.experimental.pallas import tpu as pltpu
from jax.experimental.pallas import tpu_sc as plsc

D = 64
NW = 32
B1 = 50
TC = 128
NBLK = B1 * TC // NW


def _build():
    mesh = plsc.VectorSubcoreMesh(core_axis_name="c", subcore_axis_name="s")

    @functools.partial(
        pl.kernel,
        mesh=mesh,
        out_type=jax.ShapeDtypeStruct((B1, 8, TC, 8, 128), jnp.float32),
        scratch_types=[
            pltpu.VMEM((NBLK, 128), jnp.int32),
            pltpu.VMEM((NBLK, 128), jnp.int32),
            [pltpu.VMEM((128, 128), jnp.float32) for _ in range(2)],
            [pltpu.VMEM((D, 128), jnp.float32) for _ in range(2)],
            [pltpu.SemaphoreType.DMA for _ in range(2)],
            [pltpu.SemaphoreType.DMA for _ in range(2)],
        ],
        compiler_params=pltpu.CompilerParams(
            use_tc_tiling_on_sc=True, needs_layout_passes=False
        ),
    )
    def kern(idx_hbm, idxp_hbm, table_hbm, out_hbm, idx_v, idxp_v,
             bufs, bufTs, gs, os):
        cid = lax.axis_index("c")
        sid = lax.axis_index("s")
        wid = sid * 2 + cid
        pltpu.sync_copy(idx_hbm.at[wid], idx_v)
        pltpu.sync_copy(idxp_hbm.at[wid], idxp_v)

        for b in range(2):
            pltpu.async_copy(table_hbm.at[idxp_v.at[b]], bufs[b], gs[b])

        lane = lax.iota(jnp.int32, 16)

        def wait_stores(b):
            for _ in range(8):
                pltpu.make_async_copy(
                    bufTs[b].at[pl.ds(0, 8)],
                    out_hbm.at[0].at[0].at[0],
                    os[b],
                ).wait()

        def cycle(i, _):
            for b in range(2):
                k = i * 2 + b
                bid = wid * NBLK + k
                b1 = bid >> 7
                tc = bid & 127
                pltpu.make_async_copy(
                    table_hbm.at[idxp_v.at[0]], bufs[b], gs[b]
                ).wait()
                @pl.when(k >= 2)
                def _drain():
                    wait_stores(b)

                buf = bufs[b]
                bufT = bufTs[b]
                for g in range(8):
                    toks = idx_v[k, pl.ds(g * 16, 16)]
                    row = lane + g * 16
                    half = (toks & 1) * D

                    def body(c, half):
                        vals = plsc.load_gather(buf, [row, half + c])
                        bufT[c, pl.ds(g * 16, 16)] = vals
                        return half

                    lax.fori_loop(0, D, body, half, unroll=2)
                for tr in range(8):
                    pltpu.async_copy(
                        bufT.at[pl.ds(tr * 8, 8)],
                        out_hbm.at[b1].at[tr].at[tc],
                        os[b],
                    )

                @pl.when(k + 2 < NBLK)
                def _refire():
                    pltpu.async_copy(
                        table_hbm.at[idxp_v.at[k + 2]], bufs[b], gs[b]
                    )

            return 0

        lax.fori_loop(0, NBLK // 2, cycle, 0)
        for b in range(2):
            wait_stores(b)

    return kern


def kernel(token_ids, weight):
    tid = token_ids.astype(jnp.int32).T.reshape(NW, NBLK, 128)
    w2 = weight.reshape(500000, 128)
    out6 = _build()(tid, tid >> 1, w2)
    return out6.transpose(2, 4, 0, 1, 3).reshape(16384, B1, D)


# speedup vs baseline: 1.4655x; 1.4655x over previous
import functools

import jax
import jax.numpy as jnp
from jax import lax
from jax.experimental import pallas as pl
from jax.experimental.pallas import tpu as pltpu
from jax.experimental.pallas import tpu_sc as plsc

D = 64
NW = 32
B1 = 50
TC = 128
NBLK = B1 * TC // NW


def _build():
    mesh = plsc.VectorSubcoreMesh(core_axis_name="c", subcore_axis_name="s")

    @functools.partial(
        pl.kernel,
        mesh=mesh,
        out_type=jax.ShapeDtypeStruct((B1, 8, TC, 8, 128), jnp.float32),
        scratch_types=[
            pltpu.VMEM((NBLK, 128), jnp.int32),
            pltpu.VMEM((NBLK, 128), jnp.int32),
            [pltpu.VMEM((128, 128), jnp.float32) for _ in range(2)],
            [pltpu.VMEM((D, 128), jnp.float32) for _ in range(2)],
            [pltpu.SemaphoreType.DMA for _ in range(2)],
            [pltpu.SemaphoreType.DMA for _ in range(2)],
        ],
        compiler_params=pltpu.CompilerParams(
            use_tc_tiling_on_sc=True, needs_layout_passes=False
        ),
    )
    def kern(idx_hbm, idxp_hbm, table_hbm, out_hbm, idx_v, idxp_v,
             bufs, bufTs, gs, os):
        cid = lax.axis_index("c")
        sid = lax.axis_index("s")
        wid = sid * 2 + cid
        pltpu.sync_copy(idx_hbm.at[wid], idx_v)
        pltpu.sync_copy(idxp_hbm.at[wid], idxp_v)

        for b in range(2):
            pltpu.async_copy(table_hbm.at[idxp_v.at[b]], bufs[b], gs[b])

        lane = lax.iota(jnp.int32, 16)

        def wait_stores(b):
            for _ in range(8):
                pltpu.make_async_copy(
                    bufTs[b].at[pl.ds(0, 8)],
                    out_hbm.at[0].at[0].at[0],
                    os[b],
                ).wait()

        def cycle(i, _):
            for b in range(2):
                k = i * 2 + b
                bid = wid * NBLK + k
                b1 = bid >> 7
                tc = bid & 127
                pltpu.make_async_copy(
                    table_hbm.at[idxp_v.at[0]], bufs[b], gs[b]
                ).wait()
                @pl.when(k >= 2)
                def _drain():
                    wait_stores(b)

                buf = bufs[b]
                bufT = bufTs[b]
                rows = []
                cols = []
                for g in range(8):
                    toks = idx_v[k, pl.ds(g * 16, 16)]
                    rows.append(lane + g * 16)
                    cols.append((toks & 1) * D)

                @plsc.parallel_loop(0, D, unroll=4)
                def _transpose(c):
                    for g in range(8):
                        vals = plsc.load_gather(buf, [rows[g], cols[g] + c])
                        bufT[c, pl.ds(g * 16, 16)] = vals
                for tr in range(8):
                    pltpu.async_copy(
                        bufT.at[pl.ds(tr * 8, 8)],
                        out_hbm.at[b1].at[tr].at[tc],
                        os[b],
                    )

                @pl.when(k + 2 < NBLK)
                def _refire():
                    pltpu.async_copy(
                        table_hbm.at[idxp_v.at[k + 2]], bufs[b], gs[b]
                    )

            return 0

        lax.fori_loop(0, NBLK // 2, cycle, 0)
        for b in range(2):
            wait_stores(b)

    return kern


def kernel(token_ids, weight):
    tid = token_ids.astype(jnp.int32).T.reshape(NW, NBLK, 128)
    w2 = weight.reshape(500000, 128)
    out6 = _build()(tid, tid >> 1, w2)
    return out6.transpose(2, 4, 0, 1, 3).reshape(16384, B1, D)


# speedup vs baseline: 1.4662x; 1.0004x over previous
import functools

import jax
import jax.numpy as jnp
from jax import lax
from jax.experimental import pallas as pl
from jax.experimental.pallas import tpu as pltpu
from jax.experimental.pallas import tpu_sc as plsc

D = 64
NW = 32
B1 = 50
TC = 128
NBLK = B1 * TC // NW


def _build():
    mesh = plsc.VectorSubcoreMesh(core_axis_name="c", subcore_axis_name="s")

    @functools.partial(
        pl.kernel,
        mesh=mesh,
        out_type=jax.ShapeDtypeStruct((B1, 8, TC, 8, 128), jnp.float32),
        scratch_types=[
            pltpu.VMEM((NBLK, 128), jnp.int32),
            pltpu.VMEM((NBLK, 128), jnp.int32),
            [pltpu.VMEM((128, 128), jnp.float32) for _ in range(2)],
            [pltpu.VMEM((D, 128), jnp.float32) for _ in range(2)],
            [pltpu.SemaphoreType.DMA for _ in range(2)],
            [pltpu.SemaphoreType.DMA for _ in range(2)],
        ],
        compiler_params=pltpu.CompilerParams(
            use_tc_tiling_on_sc=True, needs_layout_passes=False
        ),
    )
    def kern(idx_hbm, idxp_hbm, table_hbm, out_hbm, idx_v, idxp_v,
             bufs, bufTs, gs, os):
        cid = lax.axis_index("c")
        sid = lax.axis_index("s")
        wid = sid * 2 + cid
        pltpu.sync_copy(idx_hbm.at[wid], idx_v)
        pltpu.sync_copy(idxp_hbm.at[wid], idxp_v)

        for b in range(2):
            pltpu.async_copy(table_hbm.at[idxp_v.at[b]], bufs[b], gs[b])

        lane = lax.iota(jnp.int32, 16)

        def wait_stores(b):
            for _ in range(8):
                pltpu.make_async_copy(
                    bufTs[b].at[pl.ds(0, 8)],
                    out_hbm.at[0].at[0].at[0],
                    os[b],
                ).wait()

        def cycle(i, _):
            for b in range(2):
                k = i * 2 + b
                bid = wid * NBLK + k
                b1 = bid >> 7
                tc = bid & 127
                pltpu.make_async_copy(
                    table_hbm.at[idxp_v.at[0]], bufs[b], gs[b]
                ).wait()
                @pl.when(k >= 2)
                def _drain():
                    wait_stores(b)

                buf = bufs[b]
                bufT = bufTs[b]
                rows = []
                cols = []
                for g in range(8):
                    toks = idx_v[k, pl.ds(g * 16, 16)]
                    rows.append(lane + g * 16)
                    cols.append((toks & 1) * D)

                @plsc.parallel_loop(0, D, unroll=8)
                def _transpose(c):
                    for g in range(8):
                        vals = plsc.load_gather(buf, [rows[g], cols[g] + c])
                        bufT[c, pl.ds(g * 16, 16)] = vals
                for tr in range(8):
                    pltpu.async_copy(
                        bufT.at[pl.ds(tr * 8, 8)],
                        out_hbm.at[b1].at[tr].at[tc],
                        os[b],
                    )

                @pl.when(k + 2 < NBLK)
                def _refire():
                    pltpu.async_copy(
                        table_hbm.at[idxp_v.at[k + 2]], bufs[b], gs[b]
                    )

            return 0

        lax.fori_loop(0, NBLK // 2, cycle, 0)
        for b in range(2):
            wait_stores(b)

    return kern


def kernel(token_ids, weight):
    tid = token_ids.astype(jnp.int32).T.reshape(NW, NBLK, 128)
    w2 = weight.reshape(500000, 128)
    out6 = _build()(tid, tid >> 1, w2)
    return out6.transpose(2, 4, 0, 1, 3).reshape(16384, B1, D)


# speedup vs baseline: 1.4702x; 1.0027x over previous
import functools

import jax
import jax.numpy as jnp
from jax import lax
from jax.experimental import pallas as pl
from jax.experimental.pallas import tpu as pltpu
from jax.experimental.pallas import tpu_sc as plsc

D = 64
NW = 32
CHUNK = 128
NBUF = 8


def _build(B):
    b_w = B // NW
    nch = b_w // CHUNK
    mesh = plsc.VectorSubcoreMesh(core_axis_name="c", subcore_axis_name="s")

    @functools.partial(
        pl.kernel,
        mesh=mesh,
        out_type=jax.ShapeDtypeStruct((B, D), jnp.float32),
        scratch_types=[
            pltpu.VMEM((nch, CHUNK), jnp.int32),
            [pltpu.VMEM((CHUNK, D), jnp.float32) for _ in range(NBUF)],
            [pltpu.SemaphoreType.DMA for _ in range(NBUF)],
            [pltpu.SemaphoreType.DMA for _ in range(NBUF)],
        ],
        compiler_params=pltpu.CompilerParams(use_tc_tiling_on_sc=False),
    )
    def kern(idx_hbm, table_hbm, out_hbm, idx_v, bufs, gs, os):
        cid = lax.axis_index("c")
        sid = lax.axis_index("s")
        wid = sid * 2 + cid
        base = wid * b_w
        pltpu.sync_copy(idx_hbm.at[wid], idx_v)

        for b in range(NBUF):
            pltpu.async_copy(table_hbm.at[idx_v.at[b]], bufs[b], gs[b])

        def cycle(k, _):
            for b in range(NBUF):
                j = k * NBUF + b
                pltpu.make_async_copy(
                    table_hbm.at[idx_v.at[0]], bufs[b], gs[b]
                ).wait()
                pltpu.async_copy(
                    bufs[b], out_hbm.at[pl.ds(base + j * CHUNK, CHUNK)], os[b]
                )
            for b in range(NBUF):
                j = k * NBUF + b
                pltpu.make_async_copy(
                    bufs[b], out_hbm.at[pl.ds(base, CHUNK)], os[b]
                ).wait()

                @pl.when(j + NBUF < nch)
                def _refire():
                    pltpu.async_copy(
                        table_hbm.at[idx_v.at[j + NBUF]], bufs[b], gs[b]
                    )

            return 0

        lax.fori_loop(0, nch // NBUF, cycle, 0)

    return kern


def kernel(token_ids, weight):
    s0, s1 = token_ids.shape
    B = s0 * s1
    idx = token_ids.reshape(NW, (B // NW) // CHUNK, CHUNK).astype(jnp.int32)
    out = _build(B)(idx, weight)
    return out.reshape(s0, s1, D)
